# baseline (speedup 1.0000x reference)
import jax
import jax.numpy as jnp
from jax import lax
from jax.experimental import pallas as pl
from jax.experimental.pallas import tpu as pltpu

DIM = 1024
EMB_ROWS = 4096
R = 1720
S_ROWS = 2 * R + EMB_ROWS + 8


def _make_tc_body(nb):
    return lambda *args: _tc_body(nb, *args)


def _tc_body(nb, starts_ref, seg_first_ref, n_extra_ref, x_ref, emb_hbm,
             o_ref, tab_ref, sem):
    b = pl.program_id(0)
    rest = EMB_ROWS - (R + 8)

    @pl.when(b == 0)
    def _():
        head = pltpu.make_async_copy(
            emb_hbm.at[pl.ds(0, R + 8), :],
            tab_ref.at[pl.ds(R, R + 8), :],
            sem.at[0],
        )
        head.start()
        if nb > 1:
            pltpu.make_async_copy(
                emb_hbm.at[pl.ds(R + 8, rest), :],
                tab_ref.at[pl.ds(2 * R + 8, rest), :],
                sem.at[1],
            ).start()
        head.wait()

    if nb > 1:
        @pl.when(b == 1)
        def _():
            pltpu.make_async_copy(
                emb_hbm.at[pl.ds(R + 8, rest), :],
                tab_ref.at[pl.ds(2 * R + 8, rest), :],
                sem.at[1],
            ).wait()

    base = b * R
    s0 = seg_first_ref[b]

    off0 = base - starts_ref[s0] + R
    q0 = pl.multiple_of((off0 // 8) * 8, 8)

    def store_piece0(k):
        def f():
            big = tab_ref[pl.ds(q0, R + 8), :]
            o_ref[:, :] = big[k:k + R, :] + x_ref[:, :]
        return f

    lax.switch(off0 - q0, [store_piece0(k) for k in range(8)])

    riota = lax.broadcasted_iota(jnp.int32, (R, 1), 0)

    def piece(j, carry):
        st = starts_ref[s0 + j]
        boundary = st - base
        offj = base - st + R
        qj = pl.multiple_of((offj // 8) * 8, 8)

        def store_piecej(k):
            def f():
                big = tab_ref[pl.ds(qj, R + 8), :]
                o_ref[:, :] = jnp.where(
                    riota >= boundary,
                    big[k:k + R, :] + x_ref[:, :],
                    o_ref[:, :])
            return f

        lax.switch(offj - qj, [store_piecej(k) for k in range(8)])
        return carry

    lax.fori_loop(1, n_extra_ref[b] + 1, piece, 0)


def _tc_call(x, emb, starts, seg_first, n_extra, interpret=False):
    total = x.shape[0]
    nb = (total + R - 1) // R
    grid_spec = pltpu.PrefetchScalarGridSpec(
        num_scalar_prefetch=3,
        grid=(nb,),
        in_specs=[
            pl.BlockSpec((R, DIM), lambda b, *_: (b, 0)),
            pl.BlockSpec(memory_space=pltpu.MemorySpace.HBM),
        ],
        out_specs=pl.BlockSpec((R, DIM), lambda b, *_: (b, 0)),
        scratch_shapes=[
            pltpu.VMEM((S_ROWS, DIM), jnp.float32),
            pltpu.SemaphoreType.DMA((2,)),
        ],
    )
    return pl.pallas_call(
        _make_tc_body(nb),
        grid_spec=grid_spec,
        out_shape=jax.ShapeDtypeStruct((total, DIM), jnp.float32),
        compiler_params=pltpu.CompilerParams(
            dimension_semantics=("arbitrary",)),
        interpret=interpret,
    )(starts, seg_first, n_extra, x, emb)


def kernel(x, emb, sizes):
    total = x.shape[0]
    nb = (total + R - 1) // R
    sizes = sizes.astype(jnp.int32)
    csum = jnp.cumsum(sizes)
    starts = (csum - sizes).astype(jnp.int32)
    bstart = jnp.arange(nb, dtype=jnp.int32) * R
    seg_first = jnp.searchsorted(csum, bstart, side="right").astype(jnp.int32)
    last_row = jnp.minimum(bstart + (R - 1), total - 1)
    seg_last = jnp.searchsorted(csum, last_row, side="right").astype(jnp.int32)
    n_extra = seg_last - seg_first
    return _tc_call(x, emb, starts, seg_first, n_extra)


# speedup vs baseline: 1.0082x; 1.0082x over previous
import jax
import jax.numpy as jnp
from jax import lax
from jax.experimental import pallas as pl
from jax.experimental.pallas import tpu as pltpu

DIM = 1024
EMB_ROWS = 4096
R = 1600
S_ROWS = 2 * R + EMB_ROWS + 8


def _make_tc_body(nb):
    return lambda *args: _tc_body(nb, *args)


def _tc_body(nb, starts_ref, seg_first_ref, n_extra_ref, x_ref, emb_hbm,
             o_ref, tab_ref, sem):
    b = pl.program_id(0)
    rest = EMB_ROWS - (R + 8)

    @pl.when(b == 0)
    def _():
        head = pltpu.make_async_copy(
            emb_hbm.at[pl.ds(0, R + 8), :],
            tab_ref.at[pl.ds(R, R + 8), :],
            sem.at[0],
        )
        head.start()
        if nb > 1:
            pltpu.make_async_copy(
                emb_hbm.at[pl.ds(R + 8, rest), :],
                tab_ref.at[pl.ds(2 * R + 8, rest), :],
                sem.at[1],
            ).start()
        head.wait()

    if nb > 1:
        @pl.when(b == 1)
        def _():
            pltpu.make_async_copy(
                emb_hbm.at[pl.ds(R + 8, rest), :],
                tab_ref.at[pl.ds(2 * R + 8, rest), :],
                sem.at[1],
            ).wait()

    base = b * R
    s0 = seg_first_ref[b]

    off0 = base - starts_ref[s0] + R
    q0 = pl.multiple_of((off0 // 8) * 8, 8)

    def store_piece0(k):
        def f():
            big = tab_ref[pl.ds(q0, R + 8), :]
            o_ref[:, :] = big[k:k + R, :] + x_ref[:, :]
        return f

    lax.switch(off0 - q0, [store_piece0(k) for k in range(8)])

    riota = lax.broadcasted_iota(jnp.int32, (R, 1), 0)

    def piece(j, carry):
        st = starts_ref[s0 + j]
        boundary = st - base
        offj = base - st + R
        qj = pl.multiple_of((offj // 8) * 8, 8)

        def store_piecej(k):
            def f():
                big = tab_ref[pl.ds(qj, R + 8), :]
                o_ref[:, :] = jnp.where(
                    riota >= boundary,
                    big[k:k + R, :] + x_ref[:, :],
                    o_ref[:, :])
            return f

        lax.switch(offj - qj, [store_piecej(k) for k in range(8)])
        return carry

    lax.fori_loop(1, n_extra_ref[b] + 1, piece, 0)


def _tc_call(x, emb, starts, seg_first, n_extra, interpret=False):
    total = x.shape[0]
    nb = (total + R - 1) // R
    grid_spec = pltpu.PrefetchScalarGridSpec(
        num_scalar_prefetch=3,
        grid=(nb,),
        in_specs=[
            pl.BlockSpec((R, DIM), lambda b, *_: (b, 0)),
            pl.BlockSpec(memory_space=pltpu.MemorySpace.HBM),
        ],
        out_specs=pl.BlockSpec((R, DIM), lambda b, *_: (b, 0)),
        scratch_shapes=[
            pltpu.VMEM((S_ROWS, DIM), jnp.float32),
            pltpu.SemaphoreType.DMA((2,)),
        ],
    )
    return pl.pallas_call(
        _make_tc_body(nb),
        grid_spec=grid_spec,
        out_shape=jax.ShapeDtypeStruct((total, DIM), jnp.float32),
        compiler_params=pltpu.CompilerParams(
            dimension_semantics=("arbitrary",)),
        interpret=interpret,
    )(starts, seg_first, n_extra, x, emb)


def kernel(x, emb, sizes):
    total = x.shape[0]
    nb = (total + R - 1) // R
    sizes = sizes.astype(jnp.int32)
    csum = jnp.cumsum(sizes)
    starts = (csum - sizes).astype(jnp.int32)
    bstart = jnp.arange(nb, dtype=jnp.int32) * R
    seg_first = jnp.searchsorted(csum, bstart, side="right").astype(jnp.int32)
    last_row = jnp.minimum(bstart + (R - 1), total - 1)
    seg_last = jnp.searchsorted(csum, last_row, side="right").astype(jnp.int32)
    n_extra = seg_last - seg_first
    return _tc_call(x, emb, starts, seg_first, n_extra)


# speedup vs baseline: 1.0342x; 1.0258x over previous
import jax
import jax.numpy as jnp
from jax import lax
from jax.experimental import pallas as pl
from jax.experimental.pallas import tpu as pltpu

DIM = 1024
EMB_ROWS = 4096
R = 1664
S_ROWS = 2 * R + EMB_ROWS + 8


def _make_tc_body(nb):
    return lambda *args: _tc_body(nb, *args)


def _tc_body(nb, starts_ref, seg_first_ref, n_extra_ref, x_ref, emb_hbm,
             o_ref, tab_ref, sem):
    b = pl.program_id(0)
    rest = EMB_ROWS - (R + 8)

    @pl.when(b == 0)
    def _():
        head = pltpu.make_async_copy(
            emb_hbm.at[pl.ds(0, R + 8), :],
            tab_ref.at[pl.ds(R, R + 8), :],
            sem.at[0],
        )
        head.start()
        if nb > 1:
            pltpu.make_async_copy(
                emb_hbm.at[pl.ds(R + 8, rest), :],
                tab_ref.at[pl.ds(2 * R + 8, rest), :],
                sem.at[1],
            ).start()
        head.wait()

    if nb > 1:
        @pl.when(b == 1)
        def _():
            pltpu.make_async_copy(
                emb_hbm.at[pl.ds(R + 8, rest), :],
                tab_ref.at[pl.ds(2 * R + 8, rest), :],
                sem.at[1],
            ).wait()

    base = b * R
    s0 = seg_first_ref[b]

    off0 = base - starts_ref[s0] + R
    q0 = pl.multiple_of((off0 // 8) * 8, 8)

    def store_piece0(k):
        def f():
            big = tab_ref[pl.ds(q0, R + 8), :]
            o_ref[:, :] = big[k:k + R, :] + x_ref[:, :]
        return f

    lax.switch(off0 - q0, [store_piece0(k) for k in range(8)])

    riota = lax.broadcasted_iota(jnp.int32, (R, 1), 0)

    def piece(j, carry):
        st = starts_ref[s0 + j]
        boundary = st - base
        offj = base - st + R
        qj = pl.multiple_of((offj // 8) * 8, 8)

        def store_piecej(k):
            def f():
                big = tab_ref[pl.ds(qj, R + 8), :]
                o_ref[:, :] = jnp.where(
                    riota >= boundary,
                    big[k:k + R, :] + x_ref[:, :],
                    o_ref[:, :])
            return f

        lax.switch(offj - qj, [store_piecej(k) for k in range(8)])
        return carry

    lax.fori_loop(1, n_extra_ref[b] + 1, piece, 0)


def _tc_call(x, emb, starts, seg_first, n_extra, interpret=False):
    total = x.shape[0]
    nb = (total + R - 1) // R
    grid_spec = pltpu.PrefetchScalarGridSpec(
        num_scalar_prefetch=3,
        grid=(nb,),
        in_specs=[
            pl.BlockSpec((R, DIM), lambda b, *_: (b, 0)),
            pl.BlockSpec(memory_space=pltpu.MemorySpace.HBM),
        ],
        out_specs=pl.BlockSpec((R, DIM), lambda b, *_: (b, 0)),
        scratch_shapes=[
            pltpu.VMEM((S_ROWS, DIM), jnp.float32),
            pltpu.SemaphoreType.DMA((2,)),
        ],
    )
    return pl.pallas_call(
        _make_tc_body(nb),
        grid_spec=grid_spec,
        out_shape=jax.ShapeDtypeStruct((total, DIM), jnp.float32),
        compiler_params=pltpu.CompilerParams(
            dimension_semantics=("arbitrary",)),
        interpret=interpret,
    )(starts, seg_first, n_extra, x, emb)


def kernel(x, emb, sizes):
    total = x.shape[0]
    nb = (total + R - 1) // R
    sizes = sizes.astype(jnp.int32)
    csum = jnp.cumsum(sizes)
    starts = (csum - sizes).astype(jnp.int32)
    bstart = jnp.arange(nb, dtype=jnp.int32) * R
    seg_first = jnp.searchsorted(csum, bstart, side="right").astype(jnp.int32)
    last_row = jnp.minimum(bstart + (R - 1), total - 1)
    seg_last = jnp.searchsorted(csum, last_row, side="right").astype(jnp.int32)
    n_extra = seg_last - seg_first
    return _tc_call(x, emb, starts, seg_first, n_extra)
